# baseline (speedup 1.0000x reference)
import jax
import jax.numpy as jnp
from jax.experimental import pallas as pl

E = 320000
D = 16
G = 8
ROWS = E // G
BLK = 2000


def _mlp_kernel(a_ref, b1_ref, bias1_ref, b2_ref, bf_ref, out_ref):
    a = a_ref[...]
    h = jnp.maximum(
        jnp.dot(a, b1_ref[...], preferred_element_type=jnp.float32)
        + bias1_ref[...],
        0.0,
    )
    out_ref[...] = (
        jnp.dot(h, b2_ref[...], preferred_element_type=jnp.float32)
        + bf_ref[0, 0]
    )


def kernel(x, adjs, edge_attr, Wn, bn, We, be, Wf, bf):
    A = jnp.reshape(edge_attr.astype(jnp.float32), (ROWS, G * D))
    B1 = jnp.kron(jnp.eye(G, dtype=jnp.float32), We.astype(jnp.float32))
    bias1 = jnp.tile(be.astype(jnp.float32), (G,)).reshape(1, G * D)
    B2 = jnp.kron(jnp.eye(G, dtype=jnp.float32), Wf.astype(jnp.float32))
    bf2 = jnp.reshape(bf.astype(jnp.float32), (1, 1))

    out = pl.pallas_call(
        _mlp_kernel,
        grid=(ROWS // BLK,),
        in_specs=[
            pl.BlockSpec((BLK, G * D), lambda i: (i, 0)),
            pl.BlockSpec((G * D, G * D), lambda i: (0, 0)),
            pl.BlockSpec((1, G * D), lambda i: (0, 0)),
            pl.BlockSpec((G * D, G), lambda i: (0, 0)),
            pl.BlockSpec((1, 1), lambda i: (0, 0)),
        ],
        out_specs=pl.BlockSpec((BLK, G), lambda i: (i, 0)),
        out_shape=jax.ShapeDtypeStruct((ROWS, G), jnp.float32),
    )(A, B1, bias1, B2, bf2)

    return jnp.reshape(out, (E,))


# speedup vs baseline: 1.1673x; 1.1673x over previous
import jax
import jax.numpy as jnp
from jax.experimental import pallas as pl

E = 320000
D = 16
BLK = 12800


def _mlp_kernel(a_ref, wet_ref, be_ref, wft_ref, bf_ref, out_ref):
    at = a_ref[...].T
    h = jnp.maximum(
        jnp.dot(wet_ref[...], at, preferred_element_type=jnp.float32)
        + be_ref[...],
        0.0,
    )
    out_ref[...] = (
        jnp.dot(wft_ref[...], h, preferred_element_type=jnp.float32)
        + bf_ref[0, 0]
    )


def kernel(x, adjs, edge_attr, Wn, bn, We, be, Wf, bf):
    a = edge_attr.astype(jnp.float32)
    wet = We.astype(jnp.float32).T
    be2 = be.astype(jnp.float32).reshape(D, 1)
    wft = Wf.astype(jnp.float32).T
    bf2 = jnp.reshape(bf.astype(jnp.float32), (1, 1))

    out = pl.pallas_call(
        _mlp_kernel,
        grid=(E // BLK,),
        in_specs=[
            pl.BlockSpec((BLK, D), lambda i: (i, 0)),
            pl.BlockSpec((D, D), lambda i: (0, 0)),
            pl.BlockSpec((D, 1), lambda i: (0, 0)),
            pl.BlockSpec((1, D), lambda i: (0, 0)),
            pl.BlockSpec((1, 1), lambda i: (0, 0)),
        ],
        out_specs=pl.BlockSpec((1, BLK), lambda i: (0, i)),
        out_shape=jax.ShapeDtypeStruct((1, E), jnp.float32),
    )(a, wet, be2, wft, bf2)

    return jnp.reshape(out, (E,))
